# baseline (speedup 1.0000x reference)
import functools

import jax
import jax.numpy as jnp
from jax import lax
from jax.experimental import pallas as pl
from jax.experimental.pallas import tpu as pltpu
from jax.experimental.pallas import tpu_sc as plsc

M = 4096
N = 4096
C = 256
KNN = 8
TBLK = 512
GRID = M // TBLK


NCHK = 8
CW = N // NCHK


def _tc_topk_body(tgt_ref, spT_ref, pk_ref):
    tb = tgt_ref[...]
    dist = jnp.zeros((TBLK, N), jnp.float32)
    for d in range(3):
        diff = tb[:, d:d + 1] - spT_ref[d:d + 1, :]
        dist = dist + diff * diff
    iota = lax.broadcasted_iota(jnp.int32, (TBLK, N), 1)
    bits = lax.bitcast_convert_type(dist + 1.0, jnp.int32)
    tags = lax.shift_right_logical(iota, 9)
    T = lax.bitcast_convert_type((bits & (-8)) | tags, jnp.float32)
    iota512f = lax.broadcasted_iota(jnp.int32, (TBLK, CW), 1).astype(jnp.float32)
    vals, gidxs = [], []
    for p in range(KNN):
        f = T[:, 0:CW]
        for c in range(1, NCHK):
            f = jnp.minimum(f, T[:, c * CW:(c + 1) * CW])
        mf = jnp.min(f, axis=1, keepdims=True)
        cand = jnp.where(f == mf, iota512f, jnp.float32(1e9))
        mi = jnp.min(cand, axis=1, keepdims=True)
        mfb = lax.bitcast_convert_type(mf, jnp.int32)
        cstar = mfb & 7
        gidx = cstar * CW + mi.astype(jnp.int32)
        dval = lax.bitcast_convert_type(mfb & (-8), jnp.float32) - 1.0
        vals.append(dval)
        gidxs.append(gidx)
        if p < KNN - 1:
            T = jnp.where(iota == gidx, jnp.float32(jnp.inf), T)
    knn_d = jnp.concatenate(vals, axis=1)
    knn_i = jnp.concatenate(gidxs, axis=1)
    w = 1.0 / (knn_d + 1e-8)
    w = w / jnp.sum(w, axis=1, keepdims=True)
    wq = (w * 1048575.0 + 0.5).astype(jnp.int32)
    pk_ref[...] = jnp.bitwise_or(lax.shift_left(wq, 12), knn_i)


NGRP = 2
MG = M // NGRP

GSZ = (2560, 1536)
assert sum(GSZ) == M and all(g % 512 == 0 for g in GSZ)


def _topk_weights(tgt8_grp, spT8, mg):
    return pl.pallas_call(
        _tc_topk_body,
        grid=(mg // TBLK,),
        in_specs=[
            pl.BlockSpec((TBLK, 8), lambda i: (i, 0)),
            pl.BlockSpec((8, N), lambda i: (0, 0)),
        ],
        out_specs=pl.BlockSpec((TBLK, KNN), lambda i: (i, 0)),
        out_shape=jax.ShapeDtypeStruct((mg, KNN), jnp.int32),
    )(tgt8_grp, spT8)


NUM_SC = 2
NUM_SUBCORES = 16
NW = NUM_SC * NUM_SUBCORES
RPC = 128
TPC = RPC // KNN
LANES = 16


def _sc_gather(feat, pk_flat, mg, row0, out_rows, prev=None):
    mesh = plsc.VectorSubcoreMesh(core_axis_name="c", subcore_axis_name="s")
    tpw = mg // NW
    nch = tpw // TPC
    rpw = tpw * KNN
    ppw = row0 // NW

    scratch = [
        pltpu.VMEM((rpw,), jnp.int32),
        pltpu.VMEM((rpw,), jnp.int32),
        pltpu.VMEM((2, RPC, C), jnp.float32),
        pltpu.VMEM((2, TPC, C), jnp.float32),
        pltpu.SemaphoreType.DMA,
        pltpu.SemaphoreType.DMA,
        pltpu.SemaphoreType.DMA,
        pltpu.SemaphoreType.DMA,
        pltpu.SemaphoreType.DMA,
    ]

    @functools.partial(
        pl.kernel,
        mesh=mesh,
        out_type=jax.ShapeDtypeStruct((out_rows, C), jnp.float32),
        scratch_types=scratch,
    )
    def k(*args):
        if prev is None:
            feat_hbm, pk_hbm, out_hbm, pk_v, idx_v, rows_v, outb_v, \
                sg0, sg1, so0, so1, sp = args
        else:
            feat_hbm, pk_hbm, prev_hbm, out_hbm, pk_v, idx_v, rows_v, \
                outb_v, sg0, sg1, so0, so1, sp = args
        sg = (sg0, sg1)
        so = (so0, so1)
        cid = lax.axis_index("c")
        sid = lax.axis_index("s")
        wid = sid * NUM_SC + cid
        pass_cp = None
        if prev is not None:
            pass_cp = pltpu.async_copy(
                prev_hbm.at[pl.ds(wid * ppw, ppw), :],
                out_hbm.at[pl.ds(wid * ppw, ppw), :], sp)
        base = wid * rpw
        pltpu.sync_copy(pk_hbm.at[pl.ds(base, rpw)], pk_v)

        def unpack_body(i, carry):
            v = pk_v[pl.ds(i * LANES, LANES)]
            idx_v[pl.ds(i * LANES, LANES)] = v & 4095
            return carry

        lax.fori_loop(0, rpw // LANES, unpack_body, 0)
        gather_cp = [None, None]
        gather_cp[0] = pltpu.async_copy(feat_hbm.at[idx_v.at[pl.ds(0, RPC)]],
                                        rows_v.at[0], sg[0])
        out_cp = [None, None]
        for ch in range(nch):
            b = ch % 2
            if ch + 1 < nch:
                gather_cp[1 - b] = pltpu.async_copy(
                    feat_hbm.at[idx_v.at[pl.ds((ch + 1) * RPC, RPC)]],
                    rows_v.at[1 - b], sg[1 - b])
            gather_cp[b].wait()
            if out_cp[b] is not None:
                out_cp[b].wait()

            def pair_body(tt, carry):
                pk = pk_v[pl.ds(ch * RPC + tt * 2 * KNN, 2 * KNN)]
                wpair = lax.shift_right_logical(pk, 12).astype(
                    jnp.float32) * jnp.float32(1.0 / 1048575.0)
                for half in range(2):
                    t = tt * 2 + half
                    for cc in range(C // LANES):
                        acc = jnp.zeros((LANES,), jnp.float32)
                        for j in range(KNN):
                            acc = acc + wpair[half * KNN + j] * rows_v[
                                b, t * KNN + j, pl.ds(cc * LANES, LANES)]
                        outb_v[b, t, pl.ds(cc * LANES, LANES)] = acc
                return carry

            lax.fori_loop(0, TPC // 2, pair_body, 0)
            out_cp[b] = pltpu.async_copy(
                outb_v.at[b],
                out_hbm.at[pl.ds(row0 + wid * tpw + ch * TPC, TPC), :],
                so[b])
        out_cp[0].wait()
        out_cp[1].wait()
        if pass_cp is not None:
            pass_cp.wait()

    if prev is None:
        return k(feat, pk_flat)
    return k(feat, pk_flat, prev)


def kernel(source_points, source_features, target_points):
    spT8 = jnp.zeros((8, N), jnp.float32).at[:3, :].set(source_points.T)
    tgt8 = jnp.zeros((M, 8), jnp.float32).at[:, :3].set(target_points)
    outs = []
    row0 = 0
    for g, mg in enumerate(GSZ):
        pk = _topk_weights(
            lax.slice_in_dim(tgt8, row0, row0 + mg, axis=0), spT8, mg)
        outs.append(_sc_gather(source_features, pk.reshape(mg * KNN), mg,
                               0, mg, prev=None))
        row0 += mg
    return jnp.concatenate(outs, axis=0)


# speedup vs baseline: 1.0355x; 1.0355x over previous
import functools

import jax
import jax.numpy as jnp
from jax import lax
from jax.experimental import pallas as pl
from jax.experimental.pallas import tpu as pltpu
from jax.experimental.pallas import tpu_sc as plsc

M = 4096
N = 4096
C = 256
KNN = 8
TBLK = 512
NCHK = 8
CW = N // NCHK


def _tc_topk_body(tgt_ref, spT_ref, pk_ref):
    tb = tgt_ref[...]
    dist = jnp.zeros((TBLK, N), jnp.float32)
    for d in range(3):
        diff = tb[:, d:d + 1] - spT_ref[d:d + 1, :]
        dist = dist + diff * diff
    iota = lax.broadcasted_iota(jnp.int32, (TBLK, N), 1)
    bits = lax.bitcast_convert_type(dist + 1.0, jnp.int32)
    tags = lax.shift_right_logical(iota, 9)
    T = lax.bitcast_convert_type((bits & (-8)) | tags, jnp.float32)
    iota512f = lax.broadcasted_iota(jnp.int32, (TBLK, CW), 1).astype(jnp.float32)
    vals, gidxs = [], []
    for p in range(KNN):
        f = T[:, 0:CW]
        for c in range(1, NCHK):
            f = jnp.minimum(f, T[:, c * CW:(c + 1) * CW])
        mf = jnp.min(f, axis=1, keepdims=True)
        cand = jnp.where(f == mf, iota512f, jnp.float32(1e9))
        mi = jnp.min(cand, axis=1, keepdims=True)
        mfb = lax.bitcast_convert_type(mf, jnp.int32)
        cstar = mfb & 7
        gidx = cstar * CW + mi.astype(jnp.int32)
        dval = lax.bitcast_convert_type(mfb & (-8), jnp.float32) - 1.0
        vals.append(dval)
        gidxs.append(gidx)
        if p < KNN - 1:
            T = jnp.where(iota == gidx, jnp.float32(jnp.inf), T)
    knn_d = jnp.concatenate(vals, axis=1)
    knn_i = jnp.concatenate(gidxs, axis=1)
    w = 1.0 / (knn_d + 1e-8)
    w = w / jnp.sum(w, axis=1, keepdims=True)
    wq = (w * 1048575.0 + 0.5).astype(jnp.int32)
    pk_ref[...] = jnp.bitwise_or(lax.shift_left(wq, 12), knn_i)


GSZ = (2048, 2048)
assert sum(GSZ) == M and all(g % TBLK == 0 for g in GSZ)


def _topk_weights(tgt8_grp, spT8, mg):
    return pl.pallas_call(
        _tc_topk_body,
        grid=(mg // TBLK,),
        in_specs=[
            pl.BlockSpec((TBLK, 8), lambda i: (i, 0)),
            pl.BlockSpec((8, N), lambda i: (0, 0)),
        ],
        out_specs=pl.BlockSpec((TBLK, KNN), lambda i: (i, 0)),
        out_shape=jax.ShapeDtypeStruct((mg, KNN), jnp.int32),
    )(tgt8_grp, spT8)


NUM_SC = 2
NUM_SUBCORES = 16
NW = NUM_SC * NUM_SUBCORES
RPC = 128
TPC = RPC // KNN
LANES = 16


def _sc_gather(feat, pk_flat, mg):
    mesh = plsc.VectorSubcoreMesh(core_axis_name="c", subcore_axis_name="s")
    tpw = mg // NW
    nch = tpw // TPC
    rpw = tpw * KNN

    @functools.partial(
        pl.kernel,
        mesh=mesh,
        out_type=jax.ShapeDtypeStruct((mg, C), jnp.float32),
        scratch_types=[
            pltpu.VMEM((rpw,), jnp.int32),
            pltpu.VMEM((rpw,), jnp.int32),
            pltpu.VMEM((2, RPC, C), jnp.float32),
            pltpu.VMEM((2, TPC, C), jnp.float32),
            pltpu.SemaphoreType.DMA,
            pltpu.SemaphoreType.DMA,
            pltpu.SemaphoreType.DMA,
            pltpu.SemaphoreType.DMA,
        ],
    )
    def k(feat_hbm, pk_hbm, out_hbm, pk_v, idx_v, rows_v, outb_v,
          sg0, sg1, so0, so1):
        sg = (sg0, sg1)
        so = (so0, so1)
        cid = lax.axis_index("c")
        sid = lax.axis_index("s")
        wid = sid * NUM_SC + cid
        base = wid * rpw
        pltpu.sync_copy(pk_hbm.at[pl.ds(base, rpw)], pk_v)

        def unpack_body(i, carry):
            v = pk_v[pl.ds(i * LANES, LANES)]
            idx_v[pl.ds(i * LANES, LANES)] = v & 4095
            return carry

        lax.fori_loop(0, rpw // LANES, unpack_body, 0)
        gather_cp = [None, None]
        gather_cp[0] = pltpu.async_copy(feat_hbm.at[idx_v.at[pl.ds(0, RPC)]],
                                        rows_v.at[0], sg[0])
        out_cp = [None, None]
        for ch in range(nch):
            b = ch % 2
            if ch + 1 < nch:
                gather_cp[1 - b] = pltpu.async_copy(
                    feat_hbm.at[idx_v.at[pl.ds((ch + 1) * RPC, RPC)]],
                    rows_v.at[1 - b], sg[1 - b])
            gather_cp[b].wait()
            if out_cp[b] is not None:
                out_cp[b].wait()

            def pair_body(tt, carry):
                pk = pk_v[pl.ds(ch * RPC + tt * 2 * KNN, 2 * KNN)]
                wpair = lax.shift_right_logical(pk, 12).astype(
                    jnp.float32) * jnp.float32(1.0 / 1048575.0)
                for half in range(2):
                    t = tt * 2 + half
                    for cc in range(C // LANES):
                        acc = jnp.zeros((LANES,), jnp.float32)
                        for j in range(KNN):
                            acc = acc + wpair[half * KNN + j] * rows_v[
                                b, t * KNN + j, pl.ds(cc * LANES, LANES)]
                        outb_v[b, t, pl.ds(cc * LANES, LANES)] = acc
                return carry

            lax.fori_loop(0, TPC // 2, pair_body, 0)
            out_cp[b] = pltpu.async_copy(
                outb_v.at[b],
                out_hbm.at[pl.ds(wid * tpw + ch * TPC, TPC), :],
                so[b])
        out_cp[0].wait()
        out_cp[1].wait()

    return k(feat, pk_flat)


def kernel(source_points, source_features, target_points):
    spT8 = jnp.zeros((8, N), jnp.float32).at[:3, :].set(source_points.T)
    tgt8 = jnp.zeros((M, 8), jnp.float32).at[:, :3].set(target_points)
    outs = []
    row0 = 0
    for mg in GSZ:
        pk = _topk_weights(
            lax.slice_in_dim(tgt8, row0, row0 + mg, axis=0), spT8, mg)
        outs.append(_sc_gather(source_features, pk.reshape(mg * KNN), mg))
        row0 += mg
    return jnp.concatenate(outs, axis=0)
